# baseline (speedup 1.0000x reference)
import functools

import jax
import jax.numpy as jnp
from jax import lax
from jax.experimental import pallas as pl
from jax.experimental.pallas import tpu as pltpu
from jax.experimental.pallas import tpu_sc as plsc

B = 4096
VOCAB = 100000
EMB = 64
NUM_CAT = 26
NUM_INT = 13
NF = NUM_CAT + 1

N_ROWS = B * NUM_CAT
NW = 32
ROWS_PER_W = N_ROWS // NW
CHUNK = 832
NCHUNK = ROWS_PER_W // CHUNK


def _gather_body(table_hbm, idx_hbm, out_hbm,
                 idx0, idx1, rows0, rows1, sem0, sem1):
    wid = lax.axis_index("s") * 2 + lax.axis_index("c")
    base = wid * ROWS_PER_W
    idx_v = (idx0, idx1)
    rows_v = (rows0, rows1)
    sems = (sem0, sem1)

    handles = {}

    def issue(jn, s):
        pltpu.sync_copy(idx_hbm.at[pl.ds(base + jn * CHUNK, CHUNK)], idx_v[s])
        handles[s] = pltpu.async_copy(table_hbm.at[idx_v[s]], rows_v[s], sems[s])

    def drain(jc, s):
        handles[s].wait()
        pltpu.sync_copy(rows_v[s], out_hbm.at[pl.ds(base + jc * CHUNK, CHUNK)])

    issue(0, 0)
    for j in range(NCHUNK):
        if j + 1 < NCHUNK:
            issue(j + 1, (j + 1) % 2)
        drain(j, j % 2)


@functools.cache
def _sc_gather():
    return pl.kernel(
        _gather_body,
        out_type=jax.ShapeDtypeStruct((N_ROWS, EMB), jnp.float32),
        mesh=plsc.VectorSubcoreMesh(core_axis_name="c", subcore_axis_name="s",
                                    num_cores=2, num_subcores=16),
        scratch_types=[
            pltpu.VMEM((CHUNK,), jnp.int32),
            pltpu.VMEM((CHUNK,), jnp.int32),
            pltpu.VMEM((CHUNK, EMB), jnp.float32),
            pltpu.VMEM((CHUNK, EMB), jnp.float32),
            pltpu.SemaphoreType.DMA,
            pltpu.SemaphoreType.DMA,
        ],
        compiler_params=pltpu.CompilerParams(use_tc_tiling_on_sc=False),
    )


BLK = 512


def _bmlp_body(int_ref, bW0, bb0, bW1, bb1, bW2, bb2, out_ref):
    x = int_ref[...]
    h = jnp.maximum(x @ bW0[...] + bb0[...], 0.0)
    h = jnp.maximum(h @ bW1[...] + bb1[...], 0.0)
    out_ref[...] = jnp.maximum(h @ bW2[...] + bb2[...], 0.0)


def _dense_body(cat_ref, bm_ref, tW0, tb0, tW1, tb1, tW2, tb2, out_ref):
    bm = bm_ref[...]
    conc = jnp.concatenate([cat_ref[...], bm[:, None, :]], axis=1)

    z3 = lax.dot_general(conc, conc, (((2,), (2,)), ((0,), (0,))),
                         preferred_element_type=jnp.float32)
    irow = lax.broadcasted_iota(jnp.int32, (BLK, NF, NF), 1)
    kcol = lax.broadcasted_iota(jnp.int32, (BLK, NF, NF), 2)
    z3 = jnp.where(kcol < irow, z3, 0.0)
    interaction = z3.reshape(BLK, NF * NF)

    tin = jnp.concatenate([interaction, bm], axis=1)
    h = jnp.maximum(tin @ tW0[...] + tb0[...], 0.0)
    h = jnp.maximum(h @ tW1[...] + tb1[...], 0.0)
    o = h @ tW2[...] + tb2[...]
    out_ref[...] = 1.0 / (1.0 + jnp.exp(-o))


def _full(shape):
    return pl.BlockSpec(shape, lambda i: tuple(0 for _ in shape))


_bmlp = pl.pallas_call(
    _bmlp_body,
    grid=(1,),
    in_specs=[
        pl.BlockSpec((B, NUM_INT), lambda i: (0, 0)),
        _full((NUM_INT, 512)), _full((1, 512)),
        _full((512, 256)), _full((1, 256)),
        _full((256, EMB)), _full((1, EMB)),
    ],
    out_specs=pl.BlockSpec((B, EMB), lambda i: (0, 0)),
    out_shape=jax.ShapeDtypeStruct((B, EMB), jnp.float32),
)

_dense = pl.pallas_call(
    _dense_body,
    grid=(B // BLK,),
    in_specs=[
        pl.BlockSpec((BLK, NUM_CAT, EMB), lambda i: (i, 0, 0)),
        pl.BlockSpec((BLK, EMB), lambda i: (i, 0)),
        _full((NF * NF + EMB, 512)), _full((1, 512)),
        _full((512, 256)), _full((1, 256)),
        _full((256, 1)), _full((1, 1)),
    ],
    out_specs=pl.BlockSpec((BLK, 1), lambda i: (i, 0)),
    out_shape=jax.ShapeDtypeStruct((B, 1), jnp.float32),
)


def kernel(cat_features, int_features, emb_table,
           bW0, bb0, bW1, bb1, bW2, bb2,
           tW0, tb0, tW1, tb1, tW2, tb2):
    idx = cat_features.reshape(-1).astype(jnp.int32)
    rows = _sc_gather()(emb_table, idx)
    cat_emb = rows.reshape(B, NUM_CAT, EMB)
    bm = _bmlp(int_features,
               bW0, bb0[None, :], bW1, bb1[None, :], bW2, bb2[None, :])
    out = _dense(cat_emb, bm,
                 tW0, tb0[None, :], tW1, tb1[None, :], tW2, tb2[None, :])
    return out[:, 0]


# speedup vs baseline: 1.0039x; 1.0039x over previous
import functools

import jax
import jax.numpy as jnp
from jax import lax
from jax.experimental import pallas as pl
from jax.experimental.pallas import tpu as pltpu
from jax.experimental.pallas import tpu_sc as plsc

B = 4096
VOCAB = 100000
EMB = 64
NUM_CAT = 26
NUM_INT = 13
NF = NUM_CAT + 1

N_ROWS = B * NUM_CAT
NW = 32
ROWS_PER_W = N_ROWS // NW
CHUNK = 1664
NCHUNK = ROWS_PER_W // CHUNK


def _gather_body(table_hbm, idx_hbm, out_hbm, idx_v, rows_v, sem):
    wid = lax.axis_index("s") * 2 + lax.axis_index("c")
    base = wid * ROWS_PER_W

    def step(j, _):
        off = base + j * CHUNK
        pltpu.sync_copy(idx_hbm.at[pl.ds(off, CHUNK)], idx_v)
        pltpu.async_copy(table_hbm.at[idx_v], rows_v, sem).wait()
        pltpu.sync_copy(rows_v, out_hbm.at[pl.ds(off, CHUNK)])
        return 0

    lax.fori_loop(0, NCHUNK, step, 0)


@functools.cache
def _sc_gather():
    return pl.kernel(
        _gather_body,
        out_type=jax.ShapeDtypeStruct((N_ROWS, EMB), jnp.float32),
        mesh=plsc.VectorSubcoreMesh(core_axis_name="c", subcore_axis_name="s",
                                    num_cores=2, num_subcores=16),
        scratch_types=[
            pltpu.VMEM((CHUNK,), jnp.int32),
            pltpu.VMEM((CHUNK, EMB), jnp.float32),
            pltpu.SemaphoreType.DMA,
        ],
        compiler_params=pltpu.CompilerParams(use_tc_tiling_on_sc=False),
    )


BLK = 512


def _bmlp_body(int_ref, bW0, bb0, bW1, bb1, bW2, bb2, out_ref):
    x = int_ref[...]
    h = jnp.maximum(x @ bW0[...] + bb0[...], 0.0)
    h = jnp.maximum(h @ bW1[...] + bb1[...], 0.0)
    out_ref[...] = jnp.maximum(h @ bW2[...] + bb2[...], 0.0)


def _dense_body(cat_ref, bm_ref, tW0, tb0, tW1, tb1, tW2, tb2, out_ref):
    bm = bm_ref[...]
    conc = jnp.concatenate([cat_ref[...], bm[:, None, :]], axis=1)

    z3 = lax.dot_general(conc, conc, (((2,), (2,)), ((0,), (0,))),
                         preferred_element_type=jnp.float32)
    irow = lax.broadcasted_iota(jnp.int32, (BLK, NF, NF), 1)
    kcol = lax.broadcasted_iota(jnp.int32, (BLK, NF, NF), 2)
    z3 = jnp.where(kcol < irow, z3, 0.0)
    interaction = z3.reshape(BLK, NF * NF)

    tin = jnp.concatenate([interaction, bm], axis=1)
    h = jnp.maximum(tin @ tW0[...] + tb0[...], 0.0)
    h = jnp.maximum(h @ tW1[...] + tb1[...], 0.0)
    o = h @ tW2[...] + tb2[...]
    out_ref[...] = 1.0 / (1.0 + jnp.exp(-o))


def _full(shape):
    return pl.BlockSpec(shape, lambda i: tuple(0 for _ in shape))


_bmlp = pl.pallas_call(
    _bmlp_body,
    grid=(1,),
    in_specs=[
        pl.BlockSpec((B, NUM_INT), lambda i: (0, 0)),
        _full((NUM_INT, 512)), _full((1, 512)),
        _full((512, 256)), _full((1, 256)),
        _full((256, EMB)), _full((1, EMB)),
    ],
    out_specs=pl.BlockSpec((B, EMB), lambda i: (0, 0)),
    out_shape=jax.ShapeDtypeStruct((B, EMB), jnp.float32),
)

_dense = pl.pallas_call(
    _dense_body,
    grid=(B // BLK,),
    in_specs=[
        pl.BlockSpec((BLK, NUM_CAT, EMB), lambda i: (i, 0, 0)),
        pl.BlockSpec((BLK, EMB), lambda i: (i, 0)),
        _full((NF * NF + EMB, 512)), _full((1, 512)),
        _full((512, 256)), _full((1, 256)),
        _full((256, 1)), _full((1, 1)),
    ],
    out_specs=pl.BlockSpec((BLK, 1), lambda i: (i, 0)),
    out_shape=jax.ShapeDtypeStruct((B, 1), jnp.float32),
)


def kernel(cat_features, int_features, emb_table,
           bW0, bb0, bW1, bb1, bW2, bb2,
           tW0, tb0, tW1, tb1, tW2, tb2):
    idx = cat_features.reshape(-1).astype(jnp.int32)
    rows = _sc_gather()(emb_table, idx)
    cat_emb = rows.reshape(B, NUM_CAT, EMB)
    bm = _bmlp(int_features,
               bW0, bb0[None, :], bW1, bb1[None, :], bW2, bb2[None, :])
    out = _dense(cat_emb, bm,
                 tW0, tb0[None, :], tW1, tb1[None, :], tW2, tb2[None, :])
    return out[:, 0]
